# baseline (speedup 1.0000x reference)
import jax
import jax.numpy as jnp
from jax.experimental import pallas as pl
from jax.experimental.pallas import tpu as pltpu

_H = 8
_KVH = 2
_GS = _H // _KVH
_QKB = 8
_VB = 16
_TAU = 1.0
_BQ = 512


def _nt_dot(a, b):
    return jax.lax.dot_general(a, b, (((1,), (1,)), ((), ())),
                               preferred_element_type=jnp.float32)


def _rosa_kernel(hs_ref, wq_ref, wk_ref, wv_ref, wo_ref, ve0_ref, ve1_ref,
                 out_ref, qa_ref, ka_ref, va_ref):
    qi = pl.program_id(0)
    S = hs_ref.shape[0]

    @pl.when(qi == 0)
    def _project():
        hs = hs_ref[...]
        qp = jax.nn.sigmoid(
            jnp.dot(hs, wq_ref[...], preferred_element_type=jnp.float32) / _TAU)
        kb = jax.nn.sigmoid(
            jnp.dot(hs, wk_ref[...], preferred_element_type=jnp.float32) / _TAU)
        vb = jax.nn.sigmoid(
            jnp.dot(hs, wv_ref[...], preferred_element_type=jnp.float32) / _TAU)
        ones = jnp.ones((S, 1), jnp.float32)
        z6 = jnp.zeros((S, 16 - _QKB - 2), jnp.float32)
        z15 = jnp.zeros((S, 32 - _VB - 1), jnp.float32)
        for h in range(_H):
            qa_ref[h] = jnp.concatenate(
                [qp[:, h * _QKB:(h + 1) * _QKB], ones, ones, z6],
                axis=1).astype(jnp.bfloat16)
        for g in range(_KVH):
            kbg = kb[:, g * _QKB:(g + 1) * _QKB]
            bias = -jnp.sum(kbg, axis=1, keepdims=True)
            bias_hi = bias.astype(jnp.bfloat16).astype(jnp.float32)
            bias_lo = bias - bias_hi
            ka_ref[g] = jnp.concatenate(
                [2.0 * kbg, bias_hi, bias_lo, z6], axis=1).astype(jnp.bfloat16)
            va_ref[g] = jnp.concatenate(
                [vb[:, g * _VB:(g + 1) * _VB], ones, z15],
                axis=1).astype(jnp.bfloat16)

    base = qi * _BQ
    qhs = [qa_ref[h, pl.ds(base, _BQ), :] for h in range(_H)]

    dmask = (jax.lax.broadcasted_iota(jnp.int32, (_BQ, _BQ), 1)
             <= jax.lax.broadcasted_iota(jnp.int32, (_BQ, _BQ), 0))
    def _pv(p, vd):
        p_hi = p.astype(jnp.bfloat16)
        p_lo = (p - p_hi.astype(jnp.float32)).astype(jnp.bfloat16)
        return (jnp.dot(p_hi, vd, preferred_element_type=jnp.float32) +
                jnp.dot(p_lo, vd, preferred_element_type=jnp.float32))

    acc = []
    for h in range(_H):
        g = h // _GS
        kd = ka_ref[g, pl.ds(base, _BQ), :]
        vd = va_ref[g, pl.ds(base, _BQ), :]
        p = jnp.where(dmask, jnp.exp(_nt_dot(qhs[h], kd)), 0.0)
        acc.append(_pv(p, vd))

    def body(j, carry):
        off = j * _BQ
        new = []
        for h in range(_H):
            g = h // _GS
            kd = ka_ref[g, pl.ds(off, _BQ), :]
            vd = va_ref[g, pl.ds(off, _BQ), :]
            p = jnp.exp(_nt_dot(qhs[h], kd))
            new.append(carry[h] + _pv(p, vd))
        return tuple(new)

    acc = jax.lax.fori_loop(0, qi, body, tuple(acc))

    obits = [a[:, :_VB] / a[:, _VB:_VB + 1] for a in acc]
    ob = jnp.concatenate(obits, axis=1)
    vmix = (ve0_ref[...] * (1.0 - ob) + ve1_ref[...] * ob).astype(jnp.bfloat16)
    out_ref[...] = jnp.dot(vmix, wo_ref[...],
                           preferred_element_type=jnp.float32)


def _rosa_single(hs, Wq, Wk, Wv, Wo, ve0, ve1, interpret=False):
    S, HID = hs.shape
    full = lambda shape: pl.BlockSpec(shape, lambda i: (0,) * len(shape))
    bf = jnp.bfloat16
    return pl.pallas_call(
        _rosa_kernel,
        grid=(S // _BQ,),
        in_specs=[
            full((S, HID)),
            full(Wq.shape), full(Wk.shape), full(Wv.shape), full(Wo.shape),
            full((1, _H * _VB)), full((1, _H * _VB)),
        ],
        out_specs=pl.BlockSpec((_BQ, HID), lambda i: (i, 0)),
        out_shape=jax.ShapeDtypeStruct((S, HID), jnp.float32),
        scratch_shapes=[
            pltpu.VMEM((_H, S, 16), bf),
            pltpu.VMEM((_KVH, S, 16), bf),
            pltpu.VMEM((_KVH, S, 32), bf),
        ],
        interpret=interpret,
    )(hs.astype(bf), Wq.astype(bf), Wk.astype(bf), Wv.astype(bf),
      Wo.astype(bf), ve0.reshape(1, -1), ve1.reshape(1, -1))


def kernel(hidden_states, Wq, Wk, Wv, Wo, v_emb0, v_emb1):
    B = hidden_states.shape[0]
    outs = [_rosa_single(hidden_states[b], Wq, Wk, Wv, Wo, v_emb0, v_emb1)
            for b in range(B)]
    return jnp.stack(outs, axis=0)


# speedup vs baseline: 1.2474x; 1.2474x over previous
import jax
import jax.numpy as jnp
from jax.experimental import pallas as pl
from jax.experimental.pallas import tpu as pltpu

_H = 8
_KVH = 2
_GS = _H // _KVH
_QKB = 8
_VB = 16
_TAU = 1.0
_BQ = 512


def _nt_dot(a, b):
    return jax.lax.dot_general(a, b, (((1,), (1,)), ((), ())),
                               preferred_element_type=jnp.float32)


def _rosa_kernel(hs_ref, wq_ref, wk_ref, wv_ref, wo_ref, ve0_ref, ve1_ref,
                 out_ref, qa_ref, ka_ref, va_ref):
    qi = pl.program_id(0)
    S = hs_ref.shape[0]

    @pl.when(qi == 0)
    def _project():
        hs = hs_ref[...]
        qp = jax.nn.sigmoid(
            jnp.dot(hs, wq_ref[...], preferred_element_type=jnp.float32) / _TAU)
        kb = jax.nn.sigmoid(
            jnp.dot(hs, wk_ref[...], preferred_element_type=jnp.float32) / _TAU)
        vb = jax.nn.sigmoid(
            jnp.dot(hs, wv_ref[...], preferred_element_type=jnp.float32) / _TAU)
        ones = jnp.ones((S, 1), jnp.float32)
        z6 = jnp.zeros((S, 16 - _QKB - 2), jnp.float32)
        z15 = jnp.zeros((S, 32 - _VB - 1), jnp.float32)
        for h in range(_H):
            qa_ref[h] = jnp.concatenate(
                [qp[:, h * _QKB:(h + 1) * _QKB], ones, ones, z6],
                axis=1).astype(jnp.bfloat16)
        log2e = 1.4426950408889634
        for g in range(_KVH):
            kbg = kb[:, g * _QKB:(g + 1) * _QKB]
            bias = -log2e * jnp.sum(kbg, axis=1, keepdims=True)
            bias_hi = bias.astype(jnp.bfloat16).astype(jnp.float32)
            bias_lo = bias - bias_hi
            ka_ref[g] = jnp.concatenate(
                [(2.0 * log2e) * kbg, bias_hi, bias_lo, z6],
                axis=1).astype(jnp.bfloat16)
            va_ref[g] = jnp.concatenate(
                [vb[:, g * _VB:(g + 1) * _VB], ones, z15], axis=1)

    base = qi * _BQ
    qhs = [qa_ref[h, pl.ds(base, _BQ), :] for h in range(_H)]

    dmask = (jax.lax.broadcasted_iota(jnp.int32, (_BQ, _BQ), 1)
             <= jax.lax.broadcasted_iota(jnp.int32, (_BQ, _BQ), 0))
    acc = []
    for h in range(_H):
        g = h // _GS
        kd = ka_ref[g, pl.ds(base, _BQ), :]
        vd = va_ref[g, pl.ds(base, _BQ), :]
        p = jnp.where(dmask, jnp.exp2(_nt_dot(qhs[h], kd)), 0.0)
        acc.append(jnp.dot(p, vd, preferred_element_type=jnp.float32))

    def body(j, carry):
        off = j * _BQ
        new = []
        for h in range(_H):
            g = h // _GS
            kd = ka_ref[g, pl.ds(off, _BQ), :]
            vd = va_ref[g, pl.ds(off, _BQ), :]
            p = jnp.exp2(_nt_dot(qhs[h], kd))
            new.append(carry[h] +
                       jnp.dot(p, vd, preferred_element_type=jnp.float32))
        return tuple(new)

    acc = jax.lax.fori_loop(0, qi, body, tuple(acc))

    obits = [a[:, :_VB] / a[:, _VB:_VB + 1] for a in acc]
    ob = jnp.concatenate(obits, axis=1)
    vmix = (ve0_ref[...] * (1.0 - ob) + ve1_ref[...] * ob).astype(jnp.bfloat16)
    out_ref[...] = jnp.dot(vmix, wo_ref[...],
                           preferred_element_type=jnp.float32)


def _rosa_single(hs, Wq, Wk, Wv, Wo, ve0, ve1, interpret=False):
    S, HID = hs.shape
    full = lambda shape: pl.BlockSpec(shape, lambda i: (0,) * len(shape))
    bf = jnp.bfloat16
    return pl.pallas_call(
        _rosa_kernel,
        grid=(S // _BQ,),
        in_specs=[
            full((S, HID)),
            full(Wq.shape), full(Wk.shape), full(Wv.shape), full(Wo.shape),
            full((1, _H * _VB)), full((1, _H * _VB)),
        ],
        out_specs=pl.BlockSpec((_BQ, HID), lambda i: (i, 0)),
        out_shape=jax.ShapeDtypeStruct((S, HID), jnp.float32),
        scratch_shapes=[
            pltpu.VMEM((_H, S, 16), bf),
            pltpu.VMEM((_KVH, S, 16), bf),
            pltpu.VMEM((_KVH, S, 32), jnp.float32),
        ],
        interpret=interpret,
    )(hs.astype(bf), Wq.astype(bf), Wk.astype(bf), Wv.astype(bf),
      Wo.astype(bf), ve0.reshape(1, -1), ve1.reshape(1, -1))


def kernel(hidden_states, Wq, Wk, Wv, Wo, v_emb0, v_emb1):
    B = hidden_states.shape[0]
    outs = [_rosa_single(hidden_states[b], Wq, Wk, Wv, Wo, v_emb0, v_emb1)
            for b in range(B)]
    return jnp.stack(outs, axis=0)


# speedup vs baseline: 1.5586x; 1.2495x over previous
import jax
import jax.numpy as jnp
from jax.experimental import pallas as pl

_H = 8
_KVH = 2
_GS = _H // _KVH
_QKB = 8
_VB = 16
_TAU = 1.0
_BQ = 512

_LOG2E = 1.4426950408889634


def _nt_dot(a, b):
    return jax.lax.dot_general(a, b, (((1,), (1,)), ((), ())),
                               preferred_element_type=jnp.float32)


def _rosa_kernel(hs_ref, wq_ref, wk_ref, wv_ref, wo_ref, ve0_ref, ve1_ref,
                 out_ref):
    S = hs_ref.shape[0]
    nq = S // _BQ
    bf = jnp.bfloat16

    hs = hs_ref[...]
    qd = jnp.dot(hs, wq_ref[...], preferred_element_type=jnp.float32)
    kd = jnp.dot(hs, wk_ref[...], preferred_element_type=jnp.float32)
    vd = jnp.dot(hs, wv_ref[...], preferred_element_type=jnp.float32)
    qp = jax.nn.sigmoid(qd / _TAU)
    kb = jax.nn.sigmoid(kd / _TAU)
    vb = jax.nn.sigmoid(vd / _TAU)

    ones = jnp.ones((S, 1), jnp.float32)
    z6 = jnp.zeros((S, 16 - _QKB - 2), jnp.float32)
    z15 = jnp.zeros((S, 32 - _VB - 1), jnp.float32)
    qas = [jnp.concatenate(
        [qp[:, h * _QKB:(h + 1) * _QKB], ones, ones, z6],
        axis=1).astype(bf) for h in range(_H)]
    kas, vas = [], []
    for g in range(_KVH):
        kbg = kb[:, g * _QKB:(g + 1) * _QKB]
        bias = -_LOG2E * jnp.sum(kbg, axis=1, keepdims=True)
        bias_hi = bias.astype(bf).astype(jnp.float32)
        bias_lo = bias - bias_hi
        kas.append(jnp.concatenate(
            [(2.0 * _LOG2E) * kbg, bias_hi, bias_lo, z6], axis=1).astype(bf))
        vas.append(jnp.concatenate(
            [vb[:, g * _VB:(g + 1) * _VB], ones, z15], axis=1))

    dmask = (jax.lax.broadcasted_iota(jnp.int32, (_BQ, _BQ), 1)
             <= jax.lax.broadcasted_iota(jnp.int32, (_BQ, _BQ), 0))

    for qi in range(nq):
        r0, r1 = qi * _BQ, (qi + 1) * _BQ
        obits = []
        for h in range(_H):
            g = h // _GS
            qh = qas[h][r0:r1]
            acc = None
            for j in range(qi + 1):
                c0, c1 = j * _BQ, (j + 1) * _BQ
                p = jnp.exp2(_nt_dot(qh, kas[g][c0:c1]))
                if j == qi:
                    p = jnp.where(dmask, p, 0.0)
                o = jnp.dot(p, vas[g][c0:c1],
                            preferred_element_type=jnp.float32)
                acc = o if acc is None else acc + o
            obits.append(acc[:, :_VB] / acc[:, _VB:_VB + 1])
        ob = jnp.concatenate(obits, axis=1)
        vmix = (ve0_ref[...] * (1.0 - ob) + ve1_ref[...] * ob).astype(bf)
        out_ref[r0:r1, :] = jnp.dot(vmix, wo_ref[...],
                                    preferred_element_type=jnp.float32)


def _rosa_single(hs, Wq, Wk, Wv, Wo, ve0, ve1, interpret=False):
    S, HID = hs.shape
    bf = jnp.bfloat16
    return pl.pallas_call(
        _rosa_kernel,
        out_shape=jax.ShapeDtypeStruct((S, HID), jnp.float32),
        interpret=interpret,
    )(hs.astype(bf), Wq.astype(bf), Wk.astype(bf), Wv.astype(bf),
      Wo.astype(bf), ve0.reshape(1, -1), ve1.reshape(1, -1))


def kernel(hidden_states, Wq, Wk, Wv, Wo, v_emb0, v_emb1):
    B = hidden_states.shape[0]
    outs = [_rosa_single(hidden_states[b], Wq, Wk, Wv, Wo, v_emb0, v_emb1)
            for b in range(B)]
    return jnp.stack(outs, axis=0)
